# baseline (speedup 1.0000x reference)
import functools

import jax
import jax.numpy as jnp
from jax import lax
from jax.experimental import pallas as pl
from jax.experimental.pallas import tpu as pltpu
from jax.experimental.pallas import tpu_sc as plsc

B = 16384
NS = 26
EMB = 16
ND = 13
H = 1024
DEMB = NS * EMB
VOCAB = 1000000

TOT = B * NS
NW = 32
SPW = TOT // NW
RPS = 128
NIR = SPW // RPS
WCHUNK = VOCAB // 8


def _sc_gather(idxT, deep_pl, wide_pl):
    info = plsc.get_sparse_core_info()
    nc = info.num_cores
    mesh = plsc.VectorSubcoreMesh(core_axis_name="c", subcore_axis_name="s")

    @functools.partial(
        pl.kernel,
        mesh=mesh,
        compiler_params=pltpu.CompilerParams(use_tc_tiling_on_sc=False),
        out_type=[
            jax.ShapeDtypeStruct((EMB, NW, NIR, RPS), jnp.float32),
            jax.ShapeDtypeStruct((NW, NIR, RPS), jnp.float32),
        ],
        scratch_types=[
            pltpu.VMEM((NIR, RPS), jnp.int32),
            pltpu.VMEM((NIR, RPS), jnp.float32),
            pltpu.VMEM_SHARED((VOCAB,), jnp.float32),
            pltpu.SemaphoreType.DMA,
        ],
    )
    def k(idx_hbm, deep_hbm, wide_hbm, emb_out, wide_out,
          idx_v, val_v, plane_sp, sem):
        sid = lax.axis_index("s")
        wid = sid * nc + lax.axis_index("c")

        pltpu.sync_copy(idx_hbm.at[wid], idx_v)

        def gather_all():
            def fire(r, c):
                pltpu.make_async_copy(
                    plane_sp.at[idx_v.at[r]], val_v.at[r], sem).start()
                return c

            def drain(r, c):
                pltpu.make_async_copy(
                    plane_sp.at[idx_v.at[r]], val_v.at[r], sem).wait()
                return c

            lax.fori_loop(0, NIR, fire, 0)
            lax.fori_loop(0, NIR, drain, 0)

        def plane(p, carry):
            @pl.when(sid < 8)
            def _():
                pltpu.sync_copy(
                    deep_hbm.at[p, pl.ds(sid * WCHUNK, WCHUNK)],
                    plane_sp.at[pl.ds(sid * WCHUNK, WCHUNK)])

            plsc.subcore_barrier()
            gather_all()
            pltpu.sync_copy(val_v, emb_out.at[p, wid])
            plsc.subcore_barrier()
            return carry

        lax.fori_loop(0, EMB, plane, 0)

        @pl.when(sid < 8)
        def _():
            pltpu.sync_copy(wide_hbm.at[pl.ds(sid * WCHUNK, WCHUNK)],
                            plane_sp.at[pl.ds(sid * WCHUNK, WCHUNK)])

        plsc.subcore_barrier()
        gather_all()
        pltpu.sync_copy(val_v, wide_out.at[wid])

    return k(idxT, deep_pl, wide_pl)


def _tc_mlp(xT, dT, wv, w1r, w1d, b1, w2, b2, w3, b3, bb=1024):
    grid = B // bb

    def body(x_ref, d_ref, wv_ref, w1r_ref, w1d_ref, b1_ref, w2_ref,
             b2_ref, w3_ref, b3_ref, o_ref):
        x = x_ref[...].astype(jnp.bfloat16)
        d = d_ref[...].astype(jnp.bfloat16)
        h1 = lax.dot_general(x, w1r_ref[...], (((0,), (0,)), ((), ())),
                             preferred_element_type=jnp.float32)
        h1 = h1 + lax.dot_general(d, w1d_ref[...], (((0,), (0,)), ((), ())),
                                  preferred_element_type=jnp.float32)
        h1 = jnp.maximum(h1 + b1_ref[...], 0.0).astype(jnp.bfloat16)
        h2 = lax.dot_general(h1, w2_ref[...], (((1,), (0,)), ((), ())),
                             preferred_element_type=jnp.float32)
        h2 = jnp.maximum(h2 + b2_ref[...], 0.0)
        fc3 = lax.dot_general(w3_ref[...], h2, (((1,), (1,)), ((), ())),
                              preferred_element_type=jnp.float32)
        ws = jnp.sum(wv_ref[...], axis=0, keepdims=True)
        o_ref[...] = fc3 + ws + b3_ref[...]

    return pl.pallas_call(
        body,
        grid=(grid,),
        in_specs=[
            pl.BlockSpec((DEMB, bb), lambda i: (0, i)),
            pl.BlockSpec((ND, bb), lambda i: (0, i)),
            pl.BlockSpec((NS, bb), lambda i: (0, i)),
            pl.BlockSpec((DEMB, H), lambda i: (0, 0)),
            pl.BlockSpec((ND, H), lambda i: (0, 0)),
            pl.BlockSpec((1, H), lambda i: (0, 0)),
            pl.BlockSpec((H, H), lambda i: (0, 0)),
            pl.BlockSpec((1, H), lambda i: (0, 0)),
            pl.BlockSpec((1, H), lambda i: (0, 0)),
            pl.BlockSpec((1, 1), lambda i: (0, 0)),
        ],
        out_specs=pl.BlockSpec((1, bb), lambda i: (0, i)),
        out_shape=jax.ShapeDtypeStruct((1, B), jnp.float32),
    )(xT, dT, wv, w1r, w1d, b1, w2, b2, w3, b3)


def kernel(dense_features, sparse_features, deep_table, wide_table,
           W1, b1, W2, b2, W3, b3):
    idxT = jnp.asarray(sparse_features, jnp.int32).T.reshape(NW, NIR, RPS)
    deep_pl = deep_table.T
    wide_pl = wide_table.reshape(VOCAB)

    emb4, wide3 = _sc_gather(idxT, deep_pl, wide_pl)
    xT = emb4.reshape(DEMB, B)
    wv = wide3.reshape(NS, B)
    dT = dense_features.T

    w1r = (W1[:, :DEMB].reshape(H, NS, EMB).transpose(2, 1, 0)
           .reshape(DEMB, H).astype(jnp.bfloat16))
    w1d = W1[:, DEMB:].T.astype(jnp.bfloat16)
    w2 = W2.T.astype(jnp.bfloat16)
    w3 = W3.reshape(1, H)
    b1r = b1.reshape(1, H)
    b2r = b2.reshape(1, H)
    b3r = b3.reshape(1, 1)

    out = _tc_mlp(xT, dT, wv, w1r, w1d, b1r, w2, b2r, w3, b3r)
    return out.reshape(B, 1)


# speedup vs baseline: 2.3000x; 2.3000x over previous
import functools

import jax
import jax.numpy as jnp
from jax import lax
from jax.experimental import pallas as pl
from jax.experimental.pallas import tpu as pltpu
from jax.experimental.pallas import tpu_sc as plsc

B = 16384
NS = 26
EMB = 16
ND = 13
H = 1024
DEMB = NS * EMB
VOCAB = 1000000

TOT = B * NS
RPS = 128
NROWS = TOT // RPS
NW = 32
SPW = NROWS // NW
WCHUNK = VOCAB // 8


def _sc_gather(idx2d, deep_table, wide_flat):
    info = plsc.get_sparse_core_info()
    nc = info.num_cores
    mesh = plsc.VectorSubcoreMesh(core_axis_name="c", subcore_axis_name="s")

    @functools.partial(
        pl.kernel,
        mesh=mesh,
        compiler_params=pltpu.CompilerParams(use_tc_tiling_on_sc=False),
        out_type=[
            jax.ShapeDtypeStruct((TOT, EMB), jnp.float32),
            jax.ShapeDtypeStruct((TOT,), jnp.float32),
        ],
        scratch_types=[
            pltpu.VMEM((SPW, RPS), jnp.int32),
            pltpu.VMEM((RPS, EMB), jnp.float32),
            pltpu.VMEM((RPS,), jnp.float32),
            pltpu.VMEM_SHARED((VOCAB,), jnp.float32),
            pltpu.SemaphoreType.DMA,
            pltpu.SemaphoreType.DMA,
        ],
    )
    def k(idx_hbm, deep_hbm, wide_hbm, emb_out, wide_out,
          idx_v, rows_v, wrow_v, wide_sp, sem1, sem2):
        sid = lax.axis_index("s")
        wid = sid * nc + lax.axis_index("c")

        @pl.when(sid < 8)
        def _():
            pltpu.sync_copy(wide_hbm.at[pl.ds(sid * WCHUNK, WCHUNK)],
                            wide_sp.at[pl.ds(sid * WCHUNK, WCHUNK)])

        plsc.subcore_barrier()

        pltpu.sync_copy(idx_hbm.at[pl.ds(wid * SPW, SPW), :], idx_v)

        def body(r, carry):
            g = (wid * SPW + r) * RPS
            c1 = pltpu.async_copy(deep_hbm.at[idx_v.at[r]], rows_v, sem1)
            c2 = pltpu.async_copy(wide_sp.at[idx_v.at[r]], wrow_v, sem2)
            c1.wait()
            c2.wait()
            pltpu.sync_copy(rows_v, emb_out.at[pl.ds(g, RPS), :])
            pltpu.sync_copy(wrow_v, wide_out.at[pl.ds(g, RPS)])
            return carry

        lax.fori_loop(0, SPW, body, 0)

    return k(idx2d, deep_table, wide_flat)


def _tc_mlp(xemb, dense, wide, w1e, w1d, b1, w2, b2, w3, b3, bb=1024):
    grid = B // bb

    def body(x_ref, d_ref, wv_ref, w1e_ref, w1d_ref, b1_ref, w2_ref,
             b2_ref, w3_ref, b3_ref, o_ref):
        x = x_ref[...].astype(jnp.bfloat16)
        d = d_ref[...].astype(jnp.bfloat16)
        h1 = lax.dot_general(x, w1e_ref[...], (((1,), (0,)), ((), ())),
                             preferred_element_type=jnp.float32)
        h1 = h1 + lax.dot_general(d, w1d_ref[...], (((1,), (0,)), ((), ())),
                                  preferred_element_type=jnp.float32)
        h1 = jnp.maximum(h1 + b1_ref[...], 0.0).astype(jnp.bfloat16)
        h2 = lax.dot_general(h1, w2_ref[...], (((1,), (0,)), ((), ())),
                             preferred_element_type=jnp.float32)
        h2 = jnp.maximum(h2 + b2_ref[...], 0.0)
        fc3 = jnp.sum(h2 * w3_ref[...], axis=1, keepdims=True)
        ws = jnp.sum(wv_ref[...], axis=1, keepdims=True)
        o_ref[...] = fc3 + ws + b3_ref[...]

    return pl.pallas_call(
        body,
        grid=(grid,),
        in_specs=[
            pl.BlockSpec((bb, DEMB), lambda i: (i, 0)),
            pl.BlockSpec((bb, ND), lambda i: (i, 0)),
            pl.BlockSpec((bb, NS), lambda i: (i, 0)),
            pl.BlockSpec((DEMB, H), lambda i: (0, 0)),
            pl.BlockSpec((ND, H), lambda i: (0, 0)),
            pl.BlockSpec((1, H), lambda i: (0, 0)),
            pl.BlockSpec((H, H), lambda i: (0, 0)),
            pl.BlockSpec((1, H), lambda i: (0, 0)),
            pl.BlockSpec((1, H), lambda i: (0, 0)),
            pl.BlockSpec((1, 1), lambda i: (0, 0)),
        ],
        out_specs=pl.BlockSpec((bb, 1), lambda i: (i, 0)),
        out_shape=jax.ShapeDtypeStruct((B, 1), jnp.float32),
    )(xemb, dense, wide, w1e, w1d, b1, w2, b2, w3, b3)


def kernel(dense_features, sparse_features, deep_table, wide_table,
           W1, b1, W2, b2, W3, b3):
    idx_all = jnp.asarray(sparse_features, jnp.int32).reshape(-1)
    idx2d = idx_all.reshape(NROWS, RPS)
    wide_flat = wide_table.reshape(-1)
    emb_flat, wide_vals = _sc_gather(idx2d, deep_table, wide_flat)
    xemb = emb_flat.reshape(B, DEMB)
    widev = wide_vals.reshape(B, NS)

    w1e = W1[:, :DEMB].T.astype(jnp.bfloat16)
    w1d = W1[:, DEMB:].T.astype(jnp.bfloat16)
    w2 = W2.T.astype(jnp.bfloat16)
    w3 = W3.reshape(1, H)
    b1r = b1.reshape(1, H)
    b2r = b2.reshape(1, H)
    b3r = b3.reshape(1, 1)

    return _tc_mlp(xemb, dense_features, widev, w1e, w1d, b1r, w2,
                   b2r, w3, b3r)
